# Initial kernel scaffold; baseline (speedup 1.0000x reference)
import jax
import jax.numpy as jnp
from jax.experimental import pallas as pl


def kernel(user_table, item_table, user, positive, negative, edge_index, perm_user, perm_item, user_beta, item_beta, neg_beta):
    raise NotImplementedError("write your pallas kernel here")


# speedup vs baseline: 10.2928x; 10.2928x over previous
import functools

import jax
import jax.numpy as jnp
from jax import lax
from jax.experimental import pallas as pl
from jax.experimental.pallas import tpu as pltpu
from jax.experimental.pallas import tpu_sc as plsc

NU = 25000
NI = 25000
EMB = 64
E = 400000
B = 4096
TAU = 0.2
REG_LAMBDA = 1e-4
SSL_LAMBDA = 0.1

NC = 2
NS = 16
CH = 96
NSLOT = 4
CPT = 264
NROUND = CPT // NSLOT
EP = NS * CPT * CH
NP = 25088
RPT = NP // NS
PSUB = 32
NPOST = RPT // PSUB
CH3 = 128

_MESH = plsc.VectorSubcoreMesh(core_axis_name="c", subcore_axis_name="s")
_SC_PARAMS = pltpu.CompilerParams(use_tc_tiling_on_sc=False,
                                  needs_layout_passes=False)


def _rsqrt16(x):
    i = lax.bitcast_convert_type(x, jnp.int32)
    i = jnp.int32(0x5F3759DF) - lax.shift_right_arithmetic(i, 1)
    y = lax.bitcast_convert_type(i, jnp.float32)
    for _ in range(3):
        y = y * (1.5 - 0.5 * x * y * y)
    return y


def _k1_body(eu_r, ei_r, ut_p, it_p, su, si, zu, zi,
             deg_sh, idx2, ones_v, dbuf, sbuf, rows_v):
    c = lax.axis_index("c")
    t = lax.axis_index("s")

    for q in range(CH // 16):
        ones_v[pl.ds(q * 16, 16)] = jnp.full((16,), 1.0, jnp.float32)
    for q in range(PSUB // 16):
        dbuf[pl.ds(q * 16, 16)] = jnp.zeros((16,), jnp.float32)

    def zbody(k, _):
        pltpu.sync_copy(dbuf, deg_sh.at[pl.ds(t * RPT + k * PSUB, PSUB)])
        return 0
    lax.fori_loop(0, NPOST, zbody, 0)
    plsc.subcore_barrier()

    def scatter_deg(er):
        pltpu.sync_copy(er.at[pl.ds(t * CPT, CPT)], idx2)

        def body(j, _):
            pltpu.sync_copy(ones_v, deg_sh.at[idx2.at[j]], add=True)
            return 0
        lax.fori_loop(0, CPT, body, 0)

    pl.when(c == 0)(lambda: scatter_deg(eu_r))
    pl.when(c == 1)(lambda: scatter_deg(ei_r))
    plsc.subcore_barrier()

    def post(s_out, tab, z_out):
        def kbody(k, _):
            r0 = t * RPT + k * PSUB
            pltpu.sync_copy(deg_sh.at[pl.ds(r0, PSUB)], dbuf)
            for q in range(PSUB // 16):
                d = dbuf[pl.ds(q * 16, 16)]
                sbuf[pl.ds(q * 16, 16)] = _rsqrt16(jnp.maximum(d, 1.0))
            pltpu.sync_copy(sbuf, s_out.at[pl.ds(r0, PSUB)])
            pltpu.sync_copy(tab.at[pl.ds(r0, PSUB)], rows_v)

            def gbody(g, _):
                sv16 = sbuf[pl.ds(g * 16, 16)]
                for rr in range(16):
                    sv = sv16[rr]
                    r = g * 16 + rr
                    for q in range(4):
                        rows_v[r, pl.ds(q * 16, 16)] = (
                            rows_v[r, pl.ds(q * 16, 16)] * sv)
                return 0
            lax.fori_loop(0, PSUB // 16, gbody, 0)
            pltpu.sync_copy(rows_v, z_out.at[pl.ds(r0, PSUB)])
            return 0
        lax.fori_loop(0, NPOST, kbody, 0)

    pl.when(c == 0)(lambda: post(su, ut_p, zu))
    pl.when(c == 1)(lambda: post(si, it_p, zi))


@functools.partial(
    pl.kernel,
    out_type=(
        jax.ShapeDtypeStruct((NP,), jnp.float32),
        jax.ShapeDtypeStruct((NP,), jnp.float32),
        jax.ShapeDtypeStruct((NP, EMB), jnp.float32),
        jax.ShapeDtypeStruct((NP, EMB), jnp.float32),
    ),
    mesh=_MESH,
    compiler_params=_SC_PARAMS,
    scratch_types=[
        pltpu.VMEM_SHARED((NP,), jnp.float32),
        pltpu.VMEM((CPT, CH), jnp.int32),
        pltpu.VMEM((CH,), jnp.float32),
        pltpu.VMEM((PSUB,), jnp.float32),
        pltpu.VMEM((PSUB,), jnp.float32),
        pltpu.VMEM((PSUB, EMB), jnp.float32),
    ],
)
def _k1(*args):
    _k1_body(*args)


def _k2_body(eu_r, ei_r, zu_in, zi_in, su, si, uku, znu, uki, zni,
             acc_sh, idxg, idxs, r0_v, r1_v, r2_v, r3_v, sbuf,
             isem, rs0, rs1, rs2, rs3):
    c = lax.axis_index("c")
    t = lax.axis_index("s")
    rows = (r0_v, r1_v, r2_v, r3_v)
    rsems = (rs0, rs1, rs2, rs3)

    def zrow(r, _):
        for q in range(4):
            r0_v[r, pl.ds(q * 16, 16)] = jnp.zeros((16,), jnp.float32)
        return 0
    lax.fori_loop(0, PSUB, zrow, 0)
    z32 = r0_v.at[pl.ds(0, PSUB)]

    def zcp(k, _):
        pltpu.sync_copy(z32, acc_sh.at[pl.ds(t * RPT + k * PSUB, PSUB)])
        return 0
    lax.fori_loop(0, NPOST, zcp, 0)
    plsc.subcore_barrier()

    def run(g_er, s_er, ztab, s_hbm, uk_out, zn_out):
        base = t * CPT

        def fire_idx(rnd, half):
            pltpu.async_copy(
                g_er.at[pl.ds(base + rnd * NSLOT, NSLOT)],
                idxg.at[pl.ds(half * NSLOT, NSLOT)], isem)
            pltpu.async_copy(
                s_er.at[pl.ds(base + rnd * NSLOT, NSLOT)],
                idxs.at[pl.ds(half * NSLOT, NSLOT)], isem)

        def wait_idx(rnd, half):
            pltpu.make_async_copy(
                g_er.at[pl.ds(base + rnd * NSLOT, NSLOT)],
                idxg.at[pl.ds(half * NSLOT, NSLOT)], isem).wait()
            pltpu.make_async_copy(
                s_er.at[pl.ds(base + rnd * NSLOT, NSLOT)],
                idxs.at[pl.ds(half * NSLOT, NSLOT)], isem).wait()

        fire_idx(0, 0)
        wait_idx(0, 0)
        for b in range(NSLOT):
            pltpu.async_copy(ztab.at[idxg.at[b]], rows[b], rsems[b])

        def round_body(m, _):
            p = lax.rem(m, 2)
            pn = 1 - p

            @pl.when(m + 1 < NROUND)
            def _():
                fire_idx(m + 1, pn)

            for b in range(NSLOT):
                j = m * NSLOT + b
                pltpu.make_async_copy(
                    ztab.at[idxg.at[p * NSLOT + b]], rows[b],
                    rsems[b]).wait()
                pltpu.sync_copy(rows[b], acc_sh.at[idxs.at[p * NSLOT + b]],
                                add=True)

            @pl.when(m + 1 < NROUND)
            def _():
                wait_idx(m + 1, pn)
                for b in range(NSLOT):
                    pltpu.async_copy(
                        ztab.at[idxg.at[pn * NSLOT + b]], rows[b], rsems[b])
            return 0
        lax.fori_loop(0, NROUND, round_body, 0)
        plsc.subcore_barrier()

        p_in = r0_v.at[pl.ds(0, PSUB)]
        p_out = r1_v.at[pl.ds(0, PSUB)]

        def pbody(k, _):
            r0 = t * RPT + k * PSUB
            pltpu.sync_copy(s_hbm.at[pl.ds(r0, PSUB)], sbuf)
            pltpu.sync_copy(acc_sh.at[pl.ds(r0, PSUB)], p_in)

            def gbody(g, _):
                sv16 = sbuf[pl.ds(g * 16, 16)]
                for rr in range(16):
                    sv = sv16[rr]
                    r = g * 16 + rr
                    for q in range(4):
                        v = r0_v[r, pl.ds(q * 16, 16)] * sv
                        r0_v[r, pl.ds(q * 16, 16)] = v
                        r1_v[r, pl.ds(q * 16, 16)] = v * sv
                return 0
            lax.fori_loop(0, PSUB // 16, gbody, 0)
            pltpu.sync_copy(p_in, uk_out.at[pl.ds(r0, PSUB)])
            pltpu.sync_copy(p_out, zn_out.at[pl.ds(r0, PSUB)])
            return 0
        lax.fori_loop(0, NPOST, pbody, 0)

    pl.when(c == 0)(lambda: run(ei_r, eu_r, zi_in, su, uku, znu))
    pl.when(c == 1)(lambda: run(eu_r, ei_r, zu_in, si, uki, zni))


@functools.partial(
    pl.kernel,
    out_type=(
        jax.ShapeDtypeStruct((NP, EMB), jnp.float32),
        jax.ShapeDtypeStruct((NP, EMB), jnp.float32),
        jax.ShapeDtypeStruct((NP, EMB), jnp.float32),
        jax.ShapeDtypeStruct((NP, EMB), jnp.float32),
    ),
    mesh=_MESH,
    compiler_params=_SC_PARAMS,
    scratch_types=[
        pltpu.VMEM_SHARED((NP, EMB), jnp.float32),
        pltpu.VMEM((2 * NSLOT, CH), jnp.int32),
        pltpu.VMEM((2 * NSLOT, CH), jnp.int32),
        pltpu.VMEM((CH, EMB), jnp.float32),
        pltpu.VMEM((CH, EMB), jnp.float32),
        pltpu.VMEM((CH, EMB), jnp.float32),
        pltpu.VMEM((CH, EMB), jnp.float32),
        pltpu.VMEM((PSUB,), jnp.float32),
        pltpu.SemaphoreType.DMA,
        pltpu.SemaphoreType.DMA,
        pltpu.SemaphoreType.DMA,
        pltpu.SemaphoreType.DMA,
        pltpu.SemaphoreType.DMA,
    ],
)
def _k2(*args):
    _k2_body(*args)


def _k3_body(user, pos, neg, pu, pi, u1, u2, u3, i1, i2, i3, utp, itp, g_out,
             base, idx128, idx2, r0_v, r1_v, r2_v, sem):
    c = lax.axis_index("c")
    t = lax.axis_index("s")
    w = t * NC + c
    row0 = w * CH3

    def addrows():
        def rbody(r, _):
            for q in range(4):
                r0_v[r, pl.ds(q * 16, 16)] = (
                    r0_v[r, pl.ds(q * 16, 16)]
                    + r1_v[r, pl.ds(q * 16, 16)]
                    + r2_v[r, pl.ds(q * 16, 16)])
            return 0
        lax.fori_loop(0, CH3, rbody, 0)

    def gat3(idxref, t1, t2, t3, slot):
        d1 = pltpu.async_copy(t1.at[idxref], r0_v, sem)
        d2 = pltpu.async_copy(t2.at[idxref], r1_v, sem)
        d3 = pltpu.async_copy(t3.at[idxref], r2_v, sem)
        d1.wait()
        d2.wait()
        d3.wait()
        addrows()
        pltpu.sync_copy(r0_v, g_out.at[slot, pl.ds(row0, CH3)])

    def gat1(idxref, t1, slot):
        pltpu.async_copy(t1.at[idxref], r0_v, sem).wait()
        pltpu.sync_copy(r0_v, g_out.at[slot, pl.ds(row0, CH3)])

    def compose(base_hbm, perm_hbm):
        pltpu.sync_copy(base_hbm, base)
        pltpu.sync_copy(perm_hbm.at[pl.ds(row0, CH3)], idx128)
        for k in range(CH3 // 16):
            pv = idx128[pl.ds(k * 16, 16)]
            idx2[pl.ds(k * 16, 16)] = plsc.load_gather(base, [pv])

    pltpu.sync_copy(user.at[pl.ds(row0, CH3)], idx128)
    gat3(idx128, u1, u2, u3, 0)
    gat1(idx128, utp, 6)
    pltpu.sync_copy(pos.at[pl.ds(row0, CH3)], idx128)
    gat3(idx128, i1, i2, i3, 1)
    gat1(idx128, itp, 7)
    pltpu.sync_copy(neg.at[pl.ds(row0, CH3)], idx128)
    gat3(idx128, i1, i2, i3, 2)
    gat1(idx128, itp, 8)

    compose(user, pu)
    gat3(idx2, u1, u2, u3, 3)
    compose(pos, pi)
    gat3(idx2, i1, i2, i3, 4)
    compose(neg, pi)
    gat3(idx2, i1, i2, i3, 5)


@functools.partial(
    pl.kernel,
    out_type=jax.ShapeDtypeStruct((9, B, EMB), jnp.float32),
    mesh=_MESH,
    compiler_params=_SC_PARAMS,
    scratch_types=[
        pltpu.VMEM((B,), jnp.int32),
        pltpu.VMEM((CH3,), jnp.int32),
        pltpu.VMEM((CH3,), jnp.int32),
        pltpu.VMEM((CH3, EMB), jnp.float32),
        pltpu.VMEM((CH3, EMB), jnp.float32),
        pltpu.VMEM((CH3, EMB), jnp.float32),
        pltpu.SemaphoreType.DMA,
    ],
)
def _k3(*args):
    _k3_body(*args)


_BLK = 512
_NBLK = B // _BLK


def _nrm(x):
    n = jnp.sum(x * x, axis=-1, keepdims=True)
    return x * lax.rsqrt(jnp.maximum(n, 1e-24))


def _dotT(a, b):
    return lax.dot_general(a, b, (((1,), (1,)), ((), ())),
                           precision=lax.Precision.HIGHEST,
                           preferred_element_type=jnp.float32)


def _k4_body(g_ref, bu_ref, bi_ref, nb_ref, out_ref,
             rs_u, cs_u, rs_p, cs_p, rs_e1):
    step = pl.program_id(0)
    bi = bi_ref[0, 0]
    bu = bu_ref[0, 0]

    nU2f = _nrm(g_ref[3])
    nP2f = _nrm(g_ref[4])
    nN2f = _nrm(bi * g_ref[2] + (1.0 - bi) * g_ref[5])

    rows = pl.ds(step * _BLK, _BLK)
    nUb = _nrm(g_ref[0, rows])
    nPb = _nrm(g_ref[1, rows])

    S = jnp.exp(_dotT(nUb, nU2f) / TAU)
    rs_u[0, rows] = jnp.sum(S, axis=1)
    csu = jnp.sum(S, axis=0)
    P = jnp.exp(_dotT(nPb, nP2f) / TAU)
    rs_p[0, rows] = jnp.sum(P, axis=1)
    csp = jnp.sum(P, axis=0)
    E1 = jnp.exp(_dotT(nUb, nN2f))
    rs_e1[0, rows] = jnp.sum(E1, axis=1)

    @pl.when(step == 0)
    def _():
        cs_u[...] = jnp.zeros((1, B), jnp.float32)
        cs_p[...] = jnp.zeros((1, B), jnp.float32)

    cs_u[...] = cs_u[...] + csu[None, :]
    cs_p[...] = cs_p[...] + csp[None, :]

    @pl.when(step == _NBLK - 1)
    def _():
        U, Pm, N = g_ref[0], g_ref[1], g_ref[2]
        U2, P2 = g_ref[3], g_ref[4]
        nUf, nPf = _nrm(U), _nrm(Pm)
        nU2a, nP2a = _nrm(U2), _nrm(P2)
        nCU = _nrm(bu * U + (1.0 - bu) * U2)
        nCI = _nrm(bi * Pm + (1.0 - bi) * P2)

        nb = nb_ref[...]
        mix_u = jnp.dot(nb, U, precision=lax.Precision.HIGHEST,
                        preferred_element_type=jnp.float32)
        mix_p = jnp.dot(nb, Pm, precision=lax.Precision.HIGHEST,
                        preferred_element_type=jnp.float32)
        nMU = _nrm(mix_u)
        nMP = _nrm(mix_p)

        xcu = jnp.exp(_dotT(nUf, nMU)[:, 0] / TAU)
        xcu2 = jnp.exp(_dotT(nU2a, nMU)[:, 0] / TAU)
        xcp = jnp.exp(_dotT(nPf, nMP)[:, 0] / TAU)
        xcp2 = jnp.exp(_dotT(nP2a, nMP)[:, 0] / TAU)

        ul = jnp.mean(jnp.log(rs_u[0, :] + xcu)
                      - jnp.sum(nUf * nCU, axis=-1) / TAU)
        ul2 = jnp.mean(jnp.log(cs_u[0, :] + xcu2)
                       - jnp.sum(nU2a * nCU, axis=-1) / TAU)
        il = jnp.mean(jnp.log(rs_p[0, :] + xcp)
                      - jnp.sum(nPf * nCI, axis=-1) / TAU)
        il2 = jnp.mean(jnp.log(cs_p[0, :] + xcp2)
                       - jnp.sum(nP2a * nCI, axis=-1) / TAU)
        ssl = SSL_LAMBDA * (bu * ul + (1.0 - bu) * ul2
                            + bi * il + (1.0 - bi) * il2)

        bpr2 = (1.0 - bi) * jnp.mean(jnp.log(rs_e1[0, :])
                                     - jnp.sum(nUf * nPf, axis=-1))

        x = jnp.sum(U * N, axis=-1) - jnp.sum(U * Pm, axis=-1)
        sp = jnp.maximum(x, 0.0) + jnp.log1p(jnp.exp(-jnp.abs(x)))
        bpr = bi * jnp.mean(sp)

        reg = REG_LAMBDA * 0.5 * (jnp.sum(g_ref[6] * g_ref[6])
                                  + jnp.sum(g_ref[7] * g_ref[7])
                                  + jnp.sum(g_ref[8] * g_ref[8])) / B

        col = lax.broadcasted_iota(jnp.int32, (8, 128), 1)
        row = lax.broadcasted_iota(jnp.int32, (8, 128), 0)
        o = jnp.where(col == 0, bpr, 0.0)
        o = jnp.where(col == 1, bpr2, o)
        o = jnp.where(col == 2, reg, o)
        o = jnp.where(col == 3, ssl, o)
        out_ref[...] = jnp.where(row == 0, o, 0.0)


def _k4(g, bu, bi, nb):
    return pl.pallas_call(
        _k4_body,
        grid=(_NBLK,),
        in_specs=[
            pl.BlockSpec((9, B, EMB), lambda i: (0, 0, 0)),
            pl.BlockSpec(memory_space=pltpu.SMEM),
            pl.BlockSpec(memory_space=pltpu.SMEM),
            pl.BlockSpec((1, B), lambda i: (0, 0)),
        ],
        out_specs=pl.BlockSpec((8, 128), lambda i: (0, 0)),
        out_shape=jax.ShapeDtypeStruct((8, 128), jnp.float32),
        scratch_shapes=[pltpu.VMEM((1, B), jnp.float32) for _ in range(5)],
    )(g, bu, bi, nb)


def kernel(user_table, item_table, user, positive, negative, edge_index,
           perm_user, perm_item, user_beta, item_beta, neg_beta):
    eu = edge_index[0]
    ei = edge_index[1]
    pad = jnp.full((EP - E,), NU, jnp.int32)
    eu_r = jnp.concatenate([eu, pad]).reshape(EP // CH, CH)
    ei_r = jnp.concatenate([ei, pad]).reshape(EP // CH, CH)
    ut_p = jnp.pad(user_table, ((0, NP - NU), (0, 0)))
    it_p = jnp.pad(item_table, ((0, NP - NI), (0, 0)))

    su, si, zu, zi = _k1(eu_r, ei_r, ut_p, it_p)
    uks = []
    for _ in range(3):
        uku, znu, uki, zni = _k2(eu_r, ei_r, zu, zi, su, si)
        uks.append((uku, uki))
        zu, zi = znu, zni

    g = _k3(user, positive, negative, perm_user, perm_item,
            uks[0][0], uks[1][0], uks[2][0],
            uks[0][1], uks[1][1], uks[2][1], ut_p, it_p)

    out8 = _k4(g, user_beta.reshape(1, 1), item_beta.reshape(1, 1),
               neg_beta.reshape(1, B))
    return out8[0, :4]


# speedup vs baseline: 11.3398x; 1.1017x over previous
import functools

import jax
import jax.numpy as jnp
from jax import lax
from jax.experimental import pallas as pl
from jax.experimental.pallas import tpu as pltpu
from jax.experimental.pallas import tpu_sc as plsc

NU = 25000
NI = 25000
EMB = 64
E = 400000
B = 4096
TAU = 0.2
REG_LAMBDA = 1e-4
SSL_LAMBDA = 0.1

NC = 2
NS = 16
CH = 128
NSLOT = 3
CPT = 198
NROUND = CPT // NSLOT
EP = NS * CPT * CH
NP = 25088
RPT = NP // NS
PSUB = 32
NPOST = RPT // PSUB
CH3 = 128

_MESH = plsc.VectorSubcoreMesh(core_axis_name="c", subcore_axis_name="s")
_SC_PARAMS = pltpu.CompilerParams(use_tc_tiling_on_sc=False,
                                  needs_layout_passes=False)


def _rsqrt16(x):
    i = lax.bitcast_convert_type(x, jnp.int32)
    i = jnp.int32(0x5F3759DF) - lax.shift_right_arithmetic(i, 1)
    y = lax.bitcast_convert_type(i, jnp.float32)
    for _ in range(3):
        y = y * (1.5 - 0.5 * x * y * y)
    return y


def _k1_body(eu_r, ei_r, ut_p, it_p, su, si, zu, zi,
             deg_sh, idx2, ones_v, dbuf, sbuf, rows_v):
    c = lax.axis_index("c")
    t = lax.axis_index("s")

    for q in range(CH // 16):
        ones_v[pl.ds(q * 16, 16)] = jnp.full((16,), 1.0, jnp.float32)
    for q in range(PSUB // 16):
        dbuf[pl.ds(q * 16, 16)] = jnp.zeros((16,), jnp.float32)

    def zbody(k, _):
        pltpu.sync_copy(dbuf, deg_sh.at[pl.ds(t * RPT + k * PSUB, PSUB)])
        return 0
    lax.fori_loop(0, NPOST, zbody, 0)
    plsc.subcore_barrier()

    def scatter_deg(er):
        pltpu.sync_copy(er.at[pl.ds(t * CPT, CPT)], idx2)

        def body(j, _):
            pltpu.sync_copy(ones_v, deg_sh.at[idx2.at[j]], add=True)
            return 0
        lax.fori_loop(0, CPT, body, 0)

    pl.when(c == 0)(lambda: scatter_deg(eu_r))
    pl.when(c == 1)(lambda: scatter_deg(ei_r))
    plsc.subcore_barrier()

    def post(s_out, tab, z_out):
        def kbody(k, _):
            r0 = t * RPT + k * PSUB
            pltpu.sync_copy(deg_sh.at[pl.ds(r0, PSUB)], dbuf)
            for q in range(PSUB // 16):
                d = dbuf[pl.ds(q * 16, 16)]
                sbuf[pl.ds(q * 16, 16)] = _rsqrt16(jnp.maximum(d, 1.0))
            pltpu.sync_copy(sbuf, s_out.at[pl.ds(r0, PSUB)])
            pltpu.sync_copy(tab.at[pl.ds(r0, PSUB)], rows_v)

            def gbody(g, _):
                sv16 = sbuf[pl.ds(g * 16, 16)]
                for rr in range(16):
                    sv = sv16[rr]
                    r = g * 16 + rr
                    for q in range(4):
                        rows_v[r, pl.ds(q * 16, 16)] = (
                            rows_v[r, pl.ds(q * 16, 16)] * sv)
                return 0
            lax.fori_loop(0, PSUB // 16, gbody, 0)
            pltpu.sync_copy(rows_v, z_out.at[pl.ds(r0, PSUB)])
            return 0
        lax.fori_loop(0, NPOST, kbody, 0)

    pl.when(c == 0)(lambda: post(su, ut_p, zu))
    pl.when(c == 1)(lambda: post(si, it_p, zi))


@functools.partial(
    pl.kernel,
    out_type=(
        jax.ShapeDtypeStruct((NP,), jnp.float32),
        jax.ShapeDtypeStruct((NP,), jnp.float32),
        jax.ShapeDtypeStruct((NP, EMB), jnp.float32),
        jax.ShapeDtypeStruct((NP, EMB), jnp.float32),
    ),
    mesh=_MESH,
    compiler_params=_SC_PARAMS,
    scratch_types=[
        pltpu.VMEM_SHARED((NP,), jnp.float32),
        pltpu.VMEM((CPT, CH), jnp.int32),
        pltpu.VMEM((CH,), jnp.float32),
        pltpu.VMEM((PSUB,), jnp.float32),
        pltpu.VMEM((PSUB,), jnp.float32),
        pltpu.VMEM((PSUB, EMB), jnp.float32),
    ],
)
def _k1(*args):
    _k1_body(*args)


def _k2_body(eu_r, ei_r, zu_in, zi_in, su, si, uku, znu, uki, zni,
             acc_sh, idxg, idxs, r0_v, r1_v, r2_v, sbuf,
             isem, rs0, rs1, rs2, ss0, ss1, ss2):
    c = lax.axis_index("c")
    t = lax.axis_index("s")
    rows = (r0_v, r1_v, r2_v)
    rsems = (rs0, rs1, rs2)
    ssems = (ss0, ss1, ss2)

    def zrow(r, _):
        for q in range(4):
            r0_v[r, pl.ds(q * 16, 16)] = jnp.zeros((16,), jnp.float32)
        return 0
    lax.fori_loop(0, PSUB, zrow, 0)
    z32 = r0_v.at[pl.ds(0, PSUB)]

    def zcp(k, _):
        pltpu.sync_copy(z32, acc_sh.at[pl.ds(t * RPT + k * PSUB, PSUB)])
        return 0
    lax.fori_loop(0, NPOST, zcp, 0)
    plsc.subcore_barrier()

    def run(g_er, s_er, ztab, s_hbm, uk_out, zn_out):
        base = t * CPT

        def fire_idx(rnd, half):
            pltpu.async_copy(
                g_er.at[pl.ds(base + rnd * NSLOT, NSLOT)],
                idxg.at[pl.ds(half * NSLOT, NSLOT)], isem)
            pltpu.async_copy(
                s_er.at[pl.ds(base + rnd * NSLOT, NSLOT)],
                idxs.at[pl.ds(half * NSLOT, NSLOT)], isem)

        def wait_idx(rnd, half):
            pltpu.make_async_copy(
                g_er.at[pl.ds(base + rnd * NSLOT, NSLOT)],
                idxg.at[pl.ds(half * NSLOT, NSLOT)], isem).wait()
            pltpu.make_async_copy(
                s_er.at[pl.ds(base + rnd * NSLOT, NSLOT)],
                idxs.at[pl.ds(half * NSLOT, NSLOT)], isem).wait()

        fire_idx(0, 0)
        wait_idx(0, 0)
        for b in range(NSLOT):
            pltpu.async_copy(ztab.at[idxg.at[b]], rows[b], rsems[b])

        def round_body(m, _):
            p = lax.rem(m, 2)
            pn = 1 - p

            @pl.when(m + 1 < NROUND)
            def _():
                fire_idx(m + 1, pn)

            for b in range(NSLOT):
                pltpu.make_async_copy(
                    ztab.at[idxg.at[p * NSLOT + b]], rows[b],
                    rsems[b]).wait()
                pltpu.async_copy(rows[b], acc_sh.at[idxs.at[p * NSLOT + b]],
                                 ssems[b], add=True)

            @pl.when(m + 1 < NROUND)
            def _():
                wait_idx(m + 1, pn)
                for b in range(NSLOT):
                    pltpu.make_async_copy(
                        rows[b], acc_sh.at[idxs.at[p * NSLOT + b]],
                        ssems[b]).wait()
                    pltpu.async_copy(
                        ztab.at[idxg.at[pn * NSLOT + b]], rows[b], rsems[b])

            @pl.when(m + 1 >= NROUND)
            def _():
                for b in range(NSLOT):
                    pltpu.make_async_copy(
                        rows[b], acc_sh.at[idxs.at[p * NSLOT + b]],
                        ssems[b]).wait()
            return 0
        lax.fori_loop(0, NROUND, round_body, 0)
        plsc.subcore_barrier()

        p_in = r0_v.at[pl.ds(0, PSUB)]
        p_out = r1_v.at[pl.ds(0, PSUB)]

        def pbody(k, _):
            r0 = t * RPT + k * PSUB
            pltpu.sync_copy(s_hbm.at[pl.ds(r0, PSUB)], sbuf)
            pltpu.sync_copy(acc_sh.at[pl.ds(r0, PSUB)], p_in)

            def gbody(g, _):
                sv16 = sbuf[pl.ds(g * 16, 16)]
                for rr in range(16):
                    sv = sv16[rr]
                    r = g * 16 + rr
                    for q in range(4):
                        v = r0_v[r, pl.ds(q * 16, 16)] * sv
                        r0_v[r, pl.ds(q * 16, 16)] = v
                        r1_v[r, pl.ds(q * 16, 16)] = v * sv
                return 0
            lax.fori_loop(0, PSUB // 16, gbody, 0)
            pltpu.sync_copy(p_in, uk_out.at[pl.ds(r0, PSUB)])
            pltpu.sync_copy(p_out, zn_out.at[pl.ds(r0, PSUB)])
            return 0
        lax.fori_loop(0, NPOST, pbody, 0)

    pl.when(c == 0)(lambda: run(ei_r, eu_r, zi_in, su, uku, znu))
    pl.when(c == 1)(lambda: run(eu_r, ei_r, zu_in, si, uki, zni))


@functools.partial(
    pl.kernel,
    out_type=(
        jax.ShapeDtypeStruct((NP, EMB), jnp.float32),
        jax.ShapeDtypeStruct((NP, EMB), jnp.float32),
        jax.ShapeDtypeStruct((NP, EMB), jnp.float32),
        jax.ShapeDtypeStruct((NP, EMB), jnp.float32),
    ),
    mesh=_MESH,
    compiler_params=_SC_PARAMS,
    scratch_types=[
        pltpu.VMEM_SHARED((NP, EMB), jnp.float32),
        pltpu.VMEM((2 * NSLOT, CH), jnp.int32),
        pltpu.VMEM((2 * NSLOT, CH), jnp.int32),
        pltpu.VMEM((CH, EMB), jnp.float32),
        pltpu.VMEM((CH, EMB), jnp.float32),
        pltpu.VMEM((CH, EMB), jnp.float32),
        pltpu.VMEM((PSUB,), jnp.float32),
        pltpu.SemaphoreType.DMA,
        pltpu.SemaphoreType.DMA,
        pltpu.SemaphoreType.DMA,
        pltpu.SemaphoreType.DMA,
        pltpu.SemaphoreType.DMA,
        pltpu.SemaphoreType.DMA,
        pltpu.SemaphoreType.DMA,
    ],
)
def _k2(*args):
    _k2_body(*args)


def _k3_body(user, pos, neg, pu, pi, u1, u2, u3, i1, i2, i3, utp, itp, g_out,
             base, idx128, idx2, r0_v, r1_v, r2_v, sem):
    c = lax.axis_index("c")
    t = lax.axis_index("s")
    w = t * NC + c
    row0 = w * CH3

    def addrows():
        def rbody(r, _):
            for q in range(4):
                r0_v[r, pl.ds(q * 16, 16)] = (
                    r0_v[r, pl.ds(q * 16, 16)]
                    + r1_v[r, pl.ds(q * 16, 16)]
                    + r2_v[r, pl.ds(q * 16, 16)])
            return 0
        lax.fori_loop(0, CH3, rbody, 0)

    def gat3(idxref, t1, t2, t3, slot):
        d1 = pltpu.async_copy(t1.at[idxref], r0_v, sem)
        d2 = pltpu.async_copy(t2.at[idxref], r1_v, sem)
        d3 = pltpu.async_copy(t3.at[idxref], r2_v, sem)
        d1.wait()
        d2.wait()
        d3.wait()
        addrows()
        pltpu.sync_copy(r0_v, g_out.at[slot, pl.ds(row0, CH3)])

    def gat1(idxref, t1, slot):
        pltpu.async_copy(t1.at[idxref], r0_v, sem).wait()
        pltpu.sync_copy(r0_v, g_out.at[slot, pl.ds(row0, CH3)])

    def compose(base_hbm, perm_hbm):
        pltpu.sync_copy(base_hbm, base)
        pltpu.sync_copy(perm_hbm.at[pl.ds(row0, CH3)], idx128)
        for k in range(CH3 // 16):
            pv = idx128[pl.ds(k * 16, 16)]
            idx2[pl.ds(k * 16, 16)] = plsc.load_gather(base, [pv])

    pltpu.sync_copy(user.at[pl.ds(row0, CH3)], idx128)
    gat3(idx128, u1, u2, u3, 0)
    gat1(idx128, utp, 6)
    pltpu.sync_copy(pos.at[pl.ds(row0, CH3)], idx128)
    gat3(idx128, i1, i2, i3, 1)
    gat1(idx128, itp, 7)
    pltpu.sync_copy(neg.at[pl.ds(row0, CH3)], idx128)
    gat3(idx128, i1, i2, i3, 2)
    gat1(idx128, itp, 8)

    compose(user, pu)
    gat3(idx2, u1, u2, u3, 3)
    compose(pos, pi)
    gat3(idx2, i1, i2, i3, 4)
    compose(neg, pi)
    gat3(idx2, i1, i2, i3, 5)


@functools.partial(
    pl.kernel,
    out_type=jax.ShapeDtypeStruct((9, B, EMB), jnp.float32),
    mesh=_MESH,
    compiler_params=_SC_PARAMS,
    scratch_types=[
        pltpu.VMEM((B,), jnp.int32),
        pltpu.VMEM((CH3,), jnp.int32),
        pltpu.VMEM((CH3,), jnp.int32),
        pltpu.VMEM((CH3, EMB), jnp.float32),
        pltpu.VMEM((CH3, EMB), jnp.float32),
        pltpu.VMEM((CH3, EMB), jnp.float32),
        pltpu.SemaphoreType.DMA,
    ],
)
def _k3(*args):
    _k3_body(*args)


_BLK = 512
_NBLK = B // _BLK


def _nrm(x):
    n = jnp.sum(x * x, axis=-1, keepdims=True)
    return x * lax.rsqrt(jnp.maximum(n, 1e-24))


def _dotT(a, b):
    return lax.dot_general(a, b, (((1,), (1,)), ((), ())),
                           precision=lax.Precision.HIGHEST,
                           preferred_element_type=jnp.float32)


def _k4_body(g_ref, bu_ref, bi_ref, nb_ref, out_ref,
             rs_u, cs_u, rs_p, cs_p, rs_e1):
    step = pl.program_id(0)
    bi = bi_ref[0, 0]
    bu = bu_ref[0, 0]

    nU2f = _nrm(g_ref[3])
    nP2f = _nrm(g_ref[4])
    nN2f = _nrm(bi * g_ref[2] + (1.0 - bi) * g_ref[5])

    rows = pl.ds(step * _BLK, _BLK)
    nUb = _nrm(g_ref[0, rows])
    nPb = _nrm(g_ref[1, rows])

    S = jnp.exp(_dotT(nUb, nU2f) / TAU)
    rs_u[0, rows] = jnp.sum(S, axis=1)
    csu = jnp.sum(S, axis=0)
    P = jnp.exp(_dotT(nPb, nP2f) / TAU)
    rs_p[0, rows] = jnp.sum(P, axis=1)
    csp = jnp.sum(P, axis=0)
    E1 = jnp.exp(_dotT(nUb, nN2f))
    rs_e1[0, rows] = jnp.sum(E1, axis=1)

    @pl.when(step == 0)
    def _():
        cs_u[...] = jnp.zeros((1, B), jnp.float32)
        cs_p[...] = jnp.zeros((1, B), jnp.float32)

    cs_u[...] = cs_u[...] + csu[None, :]
    cs_p[...] = cs_p[...] + csp[None, :]

    @pl.when(step == _NBLK - 1)
    def _():
        U, Pm, N = g_ref[0], g_ref[1], g_ref[2]
        U2, P2 = g_ref[3], g_ref[4]
        nUf, nPf = _nrm(U), _nrm(Pm)
        nU2a, nP2a = _nrm(U2), _nrm(P2)
        nCU = _nrm(bu * U + (1.0 - bu) * U2)
        nCI = _nrm(bi * Pm + (1.0 - bi) * P2)

        nb = nb_ref[...]
        mix_u = jnp.dot(nb, U, precision=lax.Precision.HIGHEST,
                        preferred_element_type=jnp.float32)
        mix_p = jnp.dot(nb, Pm, precision=lax.Precision.HIGHEST,
                        preferred_element_type=jnp.float32)
        nMU = _nrm(mix_u)
        nMP = _nrm(mix_p)

        xcu = jnp.exp(_dotT(nUf, nMU)[:, 0] / TAU)
        xcu2 = jnp.exp(_dotT(nU2a, nMU)[:, 0] / TAU)
        xcp = jnp.exp(_dotT(nPf, nMP)[:, 0] / TAU)
        xcp2 = jnp.exp(_dotT(nP2a, nMP)[:, 0] / TAU)

        ul = jnp.mean(jnp.log(rs_u[0, :] + xcu)
                      - jnp.sum(nUf * nCU, axis=-1) / TAU)
        ul2 = jnp.mean(jnp.log(cs_u[0, :] + xcu2)
                       - jnp.sum(nU2a * nCU, axis=-1) / TAU)
        il = jnp.mean(jnp.log(rs_p[0, :] + xcp)
                      - jnp.sum(nPf * nCI, axis=-1) / TAU)
        il2 = jnp.mean(jnp.log(cs_p[0, :] + xcp2)
                       - jnp.sum(nP2a * nCI, axis=-1) / TAU)
        ssl = SSL_LAMBDA * (bu * ul + (1.0 - bu) * ul2
                            + bi * il + (1.0 - bi) * il2)

        bpr2 = (1.0 - bi) * jnp.mean(jnp.log(rs_e1[0, :])
                                     - jnp.sum(nUf * nPf, axis=-1))

        x = jnp.sum(U * N, axis=-1) - jnp.sum(U * Pm, axis=-1)
        sp = jnp.maximum(x, 0.0) + jnp.log1p(jnp.exp(-jnp.abs(x)))
        bpr = bi * jnp.mean(sp)

        reg = REG_LAMBDA * 0.5 * (jnp.sum(g_ref[6] * g_ref[6])
                                  + jnp.sum(g_ref[7] * g_ref[7])
                                  + jnp.sum(g_ref[8] * g_ref[8])) / B

        col = lax.broadcasted_iota(jnp.int32, (8, 128), 1)
        row = lax.broadcasted_iota(jnp.int32, (8, 128), 0)
        o = jnp.where(col == 0, bpr, 0.0)
        o = jnp.where(col == 1, bpr2, o)
        o = jnp.where(col == 2, reg, o)
        o = jnp.where(col == 3, ssl, o)
        out_ref[...] = jnp.where(row == 0, o, 0.0)


def _k4(g, bu, bi, nb):
    return pl.pallas_call(
        _k4_body,
        grid=(_NBLK,),
        in_specs=[
            pl.BlockSpec((9, B, EMB), lambda i: (0, 0, 0)),
            pl.BlockSpec(memory_space=pltpu.SMEM),
            pl.BlockSpec(memory_space=pltpu.SMEM),
            pl.BlockSpec((1, B), lambda i: (0, 0)),
        ],
        out_specs=pl.BlockSpec((8, 128), lambda i: (0, 0)),
        out_shape=jax.ShapeDtypeStruct((8, 128), jnp.float32),
        scratch_shapes=[pltpu.VMEM((1, B), jnp.float32) for _ in range(5)],
    )(g, bu, bi, nb)


def kernel(user_table, item_table, user, positive, negative, edge_index,
           perm_user, perm_item, user_beta, item_beta, neg_beta):
    eu = edge_index[0]
    ei = edge_index[1]
    pad = jnp.full((EP - E,), NU, jnp.int32)
    eu_r = jnp.concatenate([eu, pad]).reshape(EP // CH, CH)
    ei_r = jnp.concatenate([ei, pad]).reshape(EP // CH, CH)
    ut_p = jnp.pad(user_table, ((0, NP - NU), (0, 0)))
    it_p = jnp.pad(item_table, ((0, NP - NI), (0, 0)))

    su, si, zu, zi = _k1(eu_r, ei_r, ut_p, it_p)
    uks = []
    for _ in range(3):
        uku, znu, uki, zni = _k2(eu_r, ei_r, zu, zi, su, si)
        uks.append((uku, uki))
        zu, zi = znu, zni

    g = _k3(user, positive, negative, perm_user, perm_item,
            uks[0][0], uks[1][0], uks[2][0],
            uks[0][1], uks[1][1], uks[2][1], ut_p, it_p)

    out8 = _k4(g, user_beta.reshape(1, 1), item_beta.reshape(1, 1),
               neg_beta.reshape(1, B))
    return out8[0, :4]


# speedup vs baseline: 20.4164x; 1.8004x over previous
import functools

import jax
import jax.numpy as jnp
from jax import lax
from jax.experimental import pallas as pl
from jax.experimental.pallas import tpu as pltpu
from jax.experimental.pallas import tpu_sc as plsc

NU = 25000
NI = 25000
EMB = 64
E = 400000
B = 4096
TAU = 0.2
REG_LAMBDA = 1e-4
SSL_LAMBDA = 0.1

NC = 2
NS = 16
CH = 128
NSLOT = 3
ECH = E // CH
CPT0 = 195
NROUND = CPT0 // NSLOT
NP = 25088
RPT = NP // NS
PSUB = 224
NPOST = RPT // PSUB
PSUB2 = 112
NPOST2 = RPT // PSUB2
CH3 = 128

_MESH = plsc.VectorSubcoreMesh(core_axis_name="c", subcore_axis_name="s")
_SC_PARAMS = pltpu.CompilerParams(use_tc_tiling_on_sc=False,
                                  needs_layout_passes=False)


def _rsqrt16(x):
    i = lax.bitcast_convert_type(x, jnp.int32)
    i = jnp.int32(0x5F3759DF) - lax.shift_right_arithmetic(i, 1)
    y = lax.bitcast_convert_type(i, jnp.float32)
    for _ in range(3):
        y = y * (1.5 - 0.5 * x * y * y)
    return y


def _k1_body(eu_r, ei_r, ut_p, it_p, su, si, zu, zi,
             deg_sh, idx2, ones_v, dbuf, sbuf, rows_v, dsem):
    c = lax.axis_index("c")
    t = lax.axis_index("s")

    for q in range(CH // 16):
        ones_v[pl.ds(q * 16, 16)] = jnp.full((16,), 1.0, jnp.float32)
    for q in range(PSUB // 16):
        dbuf[pl.ds(q * 16, 16)] = jnp.zeros((16,), jnp.float32)

    def zbody(k, _):
        pltpu.sync_copy(dbuf, deg_sh.at[pl.ds(t * RPT + k * PSUB, PSUB)])
        return 0
    lax.fori_loop(0, NPOST, zbody, 0)
    plsc.subcore_barrier()

    base = t * CPT0 + jnp.minimum(t, 5)
    cpt_t = CPT0 + jnp.where(t < 5, 1, 0)

    def scatter_deg(er):
        pl.when(t < 5)(
            lambda: pltpu.sync_copy(er.at[pl.ds(base, CPT0 + 1)], idx2))
        pl.when(t >= 5)(
            lambda: pltpu.sync_copy(er.at[pl.ds(base, CPT0)],
                                    idx2.at[pl.ds(0, CPT0)]))

        def body(j, _):
            pltpu.async_copy(ones_v, deg_sh.at[idx2.at[j]], dsem, add=True)
            return 0
        lax.fori_loop(0, cpt_t, body, 0)

        def drain(j, _):
            pltpu.make_async_copy(ones_v, deg_sh.at[idx2.at[j]], dsem).wait()
            return 0
        lax.fori_loop(0, cpt_t, drain, 0)

    pl.when(c == 0)(lambda: scatter_deg(eu_r))
    pl.when(c == 1)(lambda: scatter_deg(ei_r))
    plsc.subcore_barrier()

    NPART = NU - 15 * RPT - 6 * PSUB

    def post(s_out, tab, z_out):
        def kbody(k, _):
            r0 = t * RPT + k * PSUB
            pltpu.sync_copy(deg_sh.at[pl.ds(r0, PSUB)], dbuf)
            for q in range(PSUB // 16):
                d = dbuf[pl.ds(q * 16, 16)]
                sbuf[pl.ds(q * 16, 16)] = _rsqrt16(jnp.maximum(d, 1.0))
            pltpu.sync_copy(sbuf, s_out.at[pl.ds(r0, PSUB)])

            pl.when(r0 + PSUB <= NU)(
                lambda: pltpu.sync_copy(tab.at[pl.ds(r0, PSUB)], rows_v))

            @pl.when((r0 < NU) & (r0 + PSUB > NU))
            def _():
                pltpu.sync_copy(tab.at[pl.ds(r0, NPART)],
                                rows_v.at[pl.ds(0, NPART)])

            @pl.when(r0 + PSUB > NU)
            def _():
                lo = jnp.where(r0 >= NU, 0, NPART)

                def zr(r, _):
                    for q in range(4):
                        rows_v[r, pl.ds(q * 16, 16)] = (
                            jnp.zeros((16,), jnp.float32))
                    return 0
                lax.fori_loop(lo, PSUB, zr, 0)

            def gbody(g, _):
                sv16 = sbuf[pl.ds(g * 16, 16)]
                for rr in range(16):
                    sv = sv16[rr]
                    r = g * 16 + rr
                    for q in range(4):
                        rows_v[r, pl.ds(q * 16, 16)] = (
                            rows_v[r, pl.ds(q * 16, 16)] * sv)
                return 0
            lax.fori_loop(0, PSUB // 16, gbody, 0)
            pltpu.sync_copy(rows_v, z_out.at[pl.ds(r0, PSUB)])
            return 0
        lax.fori_loop(0, NPOST, kbody, 0)

    pl.when(c == 0)(lambda: post(su, ut_p, zu))
    pl.when(c == 1)(lambda: post(si, it_p, zi))


@functools.partial(
    pl.kernel,
    out_type=(
        jax.ShapeDtypeStruct((NP,), jnp.float32),
        jax.ShapeDtypeStruct((NP,), jnp.float32),
        jax.ShapeDtypeStruct((NP, EMB), jnp.float32),
        jax.ShapeDtypeStruct((NP, EMB), jnp.float32),
    ),
    mesh=_MESH,
    compiler_params=_SC_PARAMS,
    scratch_types=[
        pltpu.VMEM_SHARED((NP,), jnp.float32),
        pltpu.VMEM((CPT0 + 1, CH), jnp.int32),
        pltpu.VMEM((CH,), jnp.float32),
        pltpu.VMEM((PSUB,), jnp.float32),
        pltpu.VMEM((PSUB,), jnp.float32),
        pltpu.VMEM((PSUB, EMB), jnp.float32),
        pltpu.SemaphoreType.DMA,
    ],
)
def _k1(*args):
    _k1_body(*args)


def _k2_body(eu_r, ei_r, zu_in, zi_in, su, si, uku, znu, uki, zni,
             acc_sh, idxg, idxs, r0_v, r1_v, r2_v, sbuf,
             isem, rs0, rs1, rs2, ss0, ss1, ss2):
    c = lax.axis_index("c")
    t = lax.axis_index("s")
    rows = (r0_v, r1_v, r2_v)
    rsems = (rs0, rs1, rs2)
    ssems = (ss0, ss1, ss2)

    def zrow(r, _):
        for q in range(4):
            r0_v[r, pl.ds(q * 16, 16)] = jnp.zeros((16,), jnp.float32)
        return 0
    lax.fori_loop(0, PSUB2, zrow, 0)
    z32 = r0_v.at[pl.ds(0, PSUB2)]

    def zcp(k, _):
        pltpu.sync_copy(z32, acc_sh.at[pl.ds(t * RPT + k * PSUB2, PSUB2)])
        return 0
    lax.fori_loop(0, NPOST2, zcp, 0)
    plsc.subcore_barrier()

    def run(g_er, s_er, ztab, s_hbm, uk_out, zn_out):
        base = t * CPT0 + jnp.minimum(t, 5)

        def fire_idx(rnd, half):
            pltpu.async_copy(
                g_er.at[pl.ds(base + rnd * NSLOT, NSLOT)],
                idxg.at[pl.ds(half * NSLOT, NSLOT)], isem)
            pltpu.async_copy(
                s_er.at[pl.ds(base + rnd * NSLOT, NSLOT)],
                idxs.at[pl.ds(half * NSLOT, NSLOT)], isem)

        def wait_idx(rnd, half):
            pltpu.make_async_copy(
                g_er.at[pl.ds(base + rnd * NSLOT, NSLOT)],
                idxg.at[pl.ds(half * NSLOT, NSLOT)], isem).wait()
            pltpu.make_async_copy(
                s_er.at[pl.ds(base + rnd * NSLOT, NSLOT)],
                idxs.at[pl.ds(half * NSLOT, NSLOT)], isem).wait()

        fire_idx(0, 0)
        wait_idx(0, 0)
        for b in range(NSLOT):
            pltpu.async_copy(ztab.at[idxg.at[b]], rows[b], rsems[b])

        def round_body(m, _):
            p = lax.rem(m, 2)
            pn = 1 - p

            @pl.when(m + 1 < NROUND)
            def _():
                fire_idx(m + 1, pn)

            for b in range(NSLOT):
                pltpu.make_async_copy(
                    ztab.at[idxg.at[p * NSLOT + b]], rows[b],
                    rsems[b]).wait()
                pltpu.async_copy(rows[b], acc_sh.at[idxs.at[p * NSLOT + b]],
                                 ssems[b], add=True)

            @pl.when(m + 1 < NROUND)
            def _():
                wait_idx(m + 1, pn)
                for b in range(NSLOT):
                    pltpu.make_async_copy(
                        rows[b], acc_sh.at[idxs.at[p * NSLOT + b]],
                        ssems[b]).wait()
                    pltpu.async_copy(
                        ztab.at[idxg.at[pn * NSLOT + b]], rows[b], rsems[b])

            @pl.when(m + 1 >= NROUND)
            def _():
                for b in range(NSLOT):
                    pltpu.make_async_copy(
                        rows[b], acc_sh.at[idxs.at[p * NSLOT + b]],
                        ssems[b]).wait()
            return 0
        lax.fori_loop(0, NROUND, round_body, 0)

        @pl.when(t < 5)
        def _():
            jx = base + CPT0
            pltpu.sync_copy(g_er.at[jx], idxg.at[0])
            pltpu.sync_copy(s_er.at[jx], idxs.at[0])
            pltpu.async_copy(ztab.at[idxg.at[0]], r0_v, rs0)
            pltpu.make_async_copy(ztab.at[idxg.at[0]], r0_v, rs0).wait()
            pltpu.sync_copy(r0_v, acc_sh.at[idxs.at[0]], add=True)
        plsc.subcore_barrier()

        p_in = r0_v.at[pl.ds(0, PSUB2)]
        p_out = r1_v.at[pl.ds(0, PSUB2)]

        def pbody(k, _):
            r0 = t * RPT + k * PSUB2
            pltpu.sync_copy(s_hbm.at[pl.ds(r0, PSUB2)], sbuf)
            pltpu.sync_copy(acc_sh.at[pl.ds(r0, PSUB2)], p_in)

            def gbody(g, _):
                sv16 = sbuf[pl.ds(g * 16, 16)]
                for rr in range(16):
                    sv = sv16[rr]
                    r = g * 16 + rr
                    for q in range(4):
                        v = r0_v[r, pl.ds(q * 16, 16)] * sv
                        r0_v[r, pl.ds(q * 16, 16)] = v
                        r1_v[r, pl.ds(q * 16, 16)] = v * sv
                return 0
            lax.fori_loop(0, PSUB2 // 16, gbody, 0)
            pltpu.sync_copy(p_in, uk_out.at[pl.ds(r0, PSUB2)])
            pltpu.sync_copy(p_out, zn_out.at[pl.ds(r0, PSUB2)])
            return 0
        lax.fori_loop(0, NPOST2, pbody, 0)

    pl.when(c == 0)(lambda: run(ei_r, eu_r, zi_in, su, uku, znu))
    pl.when(c == 1)(lambda: run(eu_r, ei_r, zu_in, si, uki, zni))


@functools.partial(
    pl.kernel,
    out_type=(
        jax.ShapeDtypeStruct((NP, EMB), jnp.float32),
        jax.ShapeDtypeStruct((NP, EMB), jnp.float32),
        jax.ShapeDtypeStruct((NP, EMB), jnp.float32),
        jax.ShapeDtypeStruct((NP, EMB), jnp.float32),
    ),
    mesh=_MESH,
    compiler_params=_SC_PARAMS,
    scratch_types=[
        pltpu.VMEM_SHARED((NP, EMB), jnp.float32),
        pltpu.VMEM((2 * NSLOT, CH), jnp.int32),
        pltpu.VMEM((2 * NSLOT, CH), jnp.int32),
        pltpu.VMEM((CH, EMB), jnp.float32),
        pltpu.VMEM((CH, EMB), jnp.float32),
        pltpu.VMEM((CH, EMB), jnp.float32),
        pltpu.VMEM((PSUB2,), jnp.float32),
        pltpu.SemaphoreType.DMA,
        pltpu.SemaphoreType.DMA,
        pltpu.SemaphoreType.DMA,
        pltpu.SemaphoreType.DMA,
        pltpu.SemaphoreType.DMA,
        pltpu.SemaphoreType.DMA,
        pltpu.SemaphoreType.DMA,
    ],
)
def _k2(*args):
    _k2_body(*args)


def _k3_body(user, pos, neg, pu, pi, u1, u2, u3, i1, i2, i3, utp, itp, g_out,
             base, idx128, idx2, r0_v, r1_v, r2_v, sem):
    c = lax.axis_index("c")
    t = lax.axis_index("s")
    w = t * NC + c
    row0 = w * CH3

    def addrows():
        def rbody(r, _):
            for q in range(4):
                r0_v[r, pl.ds(q * 16, 16)] = (
                    r0_v[r, pl.ds(q * 16, 16)]
                    + r1_v[r, pl.ds(q * 16, 16)]
                    + r2_v[r, pl.ds(q * 16, 16)])
            return 0
        lax.fori_loop(0, CH3, rbody, 0)

    def gat3(idxref, t1, t2, t3, slot):
        d1 = pltpu.async_copy(t1.at[idxref], r0_v, sem)
        d2 = pltpu.async_copy(t2.at[idxref], r1_v, sem)
        d3 = pltpu.async_copy(t3.at[idxref], r2_v, sem)
        d1.wait()
        d2.wait()
        d3.wait()
        addrows()
        pltpu.sync_copy(r0_v, g_out.at[slot, pl.ds(row0, CH3)])

    def gat1(idxref, t1, slot):
        pltpu.async_copy(t1.at[idxref], r0_v, sem).wait()
        pltpu.sync_copy(r0_v, g_out.at[slot, pl.ds(row0, CH3)])

    def compose(base_hbm, perm_hbm):
        pltpu.sync_copy(base_hbm, base)
        pltpu.sync_copy(perm_hbm.at[pl.ds(row0, CH3)], idx128)
        for k in range(CH3 // 16):
            pv = idx128[pl.ds(k * 16, 16)]
            idx2[pl.ds(k * 16, 16)] = plsc.load_gather(base, [pv])

    pltpu.sync_copy(user.at[pl.ds(row0, CH3)], idx128)
    gat3(idx128, u1, u2, u3, 0)
    gat1(idx128, utp, 6)
    pltpu.sync_copy(pos.at[pl.ds(row0, CH3)], idx128)
    gat3(idx128, i1, i2, i3, 1)
    gat1(idx128, itp, 7)
    pltpu.sync_copy(neg.at[pl.ds(row0, CH3)], idx128)
    gat3(idx128, i1, i2, i3, 2)
    gat1(idx128, itp, 8)

    compose(user, pu)
    gat3(idx2, u1, u2, u3, 3)
    compose(pos, pi)
    gat3(idx2, i1, i2, i3, 4)
    compose(neg, pi)
    gat3(idx2, i1, i2, i3, 5)


@functools.partial(
    pl.kernel,
    out_type=jax.ShapeDtypeStruct((9, B, EMB), jnp.float32),
    mesh=_MESH,
    compiler_params=_SC_PARAMS,
    scratch_types=[
        pltpu.VMEM((B,), jnp.int32),
        pltpu.VMEM((CH3,), jnp.int32),
        pltpu.VMEM((CH3,), jnp.int32),
        pltpu.VMEM((CH3, EMB), jnp.float32),
        pltpu.VMEM((CH3, EMB), jnp.float32),
        pltpu.VMEM((CH3, EMB), jnp.float32),
        pltpu.SemaphoreType.DMA,
    ],
)
def _k3(*args):
    _k3_body(*args)


_BLK = 512
_NBLK = B // _BLK


def _nrm(x):
    n = jnp.sum(x * x, axis=-1, keepdims=True)
    return x * lax.rsqrt(jnp.maximum(n, 1e-24))


def _dotT(a, b):
    return lax.dot_general(a, b, (((1,), (1,)), ((), ())),
                           preferred_element_type=jnp.float32)


def _k4_body(g_ref, bu_ref, bi_ref, nb_ref, out_ref,
             rs_u, cs_u, rs_p, cs_p, rs_e1):
    step = pl.program_id(0)
    bi = bi_ref[0, 0]
    bu = bu_ref[0, 0]

    nU2f = _nrm(g_ref[3])
    nP2f = _nrm(g_ref[4])
    nN2f = _nrm(bi * g_ref[2] + (1.0 - bi) * g_ref[5])

    rows = pl.ds(step * _BLK, _BLK)
    nUb = _nrm(g_ref[0, rows])
    nPb = _nrm(g_ref[1, rows])

    S = jnp.exp(_dotT(nUb, nU2f) / TAU)
    rs_u[0, rows] = jnp.sum(S, axis=1)
    csu = jnp.sum(S, axis=0)
    P = jnp.exp(_dotT(nPb, nP2f) / TAU)
    rs_p[0, rows] = jnp.sum(P, axis=1)
    csp = jnp.sum(P, axis=0)
    E1 = jnp.exp(_dotT(nUb, nN2f))
    rs_e1[0, rows] = jnp.sum(E1, axis=1)

    @pl.when(step == 0)
    def _():
        cs_u[...] = jnp.zeros((1, B), jnp.float32)
        cs_p[...] = jnp.zeros((1, B), jnp.float32)

    cs_u[...] = cs_u[...] + csu[None, :]
    cs_p[...] = cs_p[...] + csp[None, :]

    @pl.when(step == _NBLK - 1)
    def _():
        U, Pm, N = g_ref[0], g_ref[1], g_ref[2]
        U2, P2 = g_ref[3], g_ref[4]
        nUf, nPf = _nrm(U), _nrm(Pm)
        nU2a, nP2a = _nrm(U2), _nrm(P2)
        nCU = _nrm(bu * U + (1.0 - bu) * U2)
        nCI = _nrm(bi * Pm + (1.0 - bi) * P2)

        nb = nb_ref[...]
        mix_u = jnp.dot(nb, U, preferred_element_type=jnp.float32)
        mix_p = jnp.dot(nb, Pm, preferred_element_type=jnp.float32)
        nMU = _nrm(mix_u)
        nMP = _nrm(mix_p)

        xcu = jnp.exp(_dotT(nUf, nMU)[:, 0] / TAU)
        xcu2 = jnp.exp(_dotT(nU2a, nMU)[:, 0] / TAU)
        xcp = jnp.exp(_dotT(nPf, nMP)[:, 0] / TAU)
        xcp2 = jnp.exp(_dotT(nP2a, nMP)[:, 0] / TAU)

        ul = jnp.mean(jnp.log(rs_u[0, :] + xcu)
                      - jnp.sum(nUf * nCU, axis=-1) / TAU)
        ul2 = jnp.mean(jnp.log(cs_u[0, :] + xcu2)
                       - jnp.sum(nU2a * nCU, axis=-1) / TAU)
        il = jnp.mean(jnp.log(rs_p[0, :] + xcp)
                      - jnp.sum(nPf * nCI, axis=-1) / TAU)
        il2 = jnp.mean(jnp.log(cs_p[0, :] + xcp2)
                       - jnp.sum(nP2a * nCI, axis=-1) / TAU)
        ssl = SSL_LAMBDA * (bu * ul + (1.0 - bu) * ul2
                            + bi * il + (1.0 - bi) * il2)

        bpr2 = (1.0 - bi) * jnp.mean(jnp.log(rs_e1[0, :])
                                     - jnp.sum(nUf * nPf, axis=-1))

        x = jnp.sum(U * N, axis=-1) - jnp.sum(U * Pm, axis=-1)
        sp = jnp.maximum(x, 0.0) + jnp.log1p(jnp.exp(-jnp.abs(x)))
        bpr = bi * jnp.mean(sp)

        reg = REG_LAMBDA * 0.5 * (jnp.sum(g_ref[6] * g_ref[6])
                                  + jnp.sum(g_ref[7] * g_ref[7])
                                  + jnp.sum(g_ref[8] * g_ref[8])) / B

        col = lax.broadcasted_iota(jnp.int32, (8, 128), 1)
        row = lax.broadcasted_iota(jnp.int32, (8, 128), 0)
        o = jnp.where(col == 0, bpr, 0.0)
        o = jnp.where(col == 1, bpr2, o)
        o = jnp.where(col == 2, reg, o)
        o = jnp.where(col == 3, ssl, o)
        out_ref[...] = jnp.where(row == 0, o, 0.0)


def _k4(g, bu, bi, nb):
    return pl.pallas_call(
        _k4_body,
        grid=(_NBLK,),
        in_specs=[
            pl.BlockSpec((9, B, EMB), lambda i: (0, 0, 0)),
            pl.BlockSpec(memory_space=pltpu.SMEM),
            pl.BlockSpec(memory_space=pltpu.SMEM),
            pl.BlockSpec((1, B), lambda i: (0, 0)),
        ],
        out_specs=pl.BlockSpec((8, 128), lambda i: (0, 0)),
        out_shape=jax.ShapeDtypeStruct((8, 128), jnp.float32),
        scratch_shapes=[pltpu.VMEM((1, B), jnp.float32) for _ in range(5)],
    )(g, bu, bi, nb)


def kernel(user_table, item_table, user, positive, negative, edge_index,
           perm_user, perm_item, user_beta, item_beta, neg_beta):
    eu_r = edge_index[0].reshape(ECH, CH)
    ei_r = edge_index[1].reshape(ECH, CH)

    su, si, zu, zi = _k1(eu_r, ei_r, user_table, item_table)
    uks = []
    for _ in range(3):
        uku, znu, uki, zni = _k2(eu_r, ei_r, zu, zi, su, si)
        uks.append((uku, uki))
        zu, zi = znu, zni

    g = _k3(user, positive, negative, perm_user, perm_item,
            uks[0][0], uks[1][0], uks[2][0],
            uks[0][1], uks[1][1], uks[2][1], user_table, item_table)

    out8 = _k4(g, user_beta.reshape(1, 1), item_beta.reshape(1, 1),
               neg_beta.reshape(1, B))
    return out8[0, :4]
